# baseline (speedup 1.0000x reference)
import functools

import jax
import jax.numpy as jnp
from jax import lax
from jax.experimental import pallas as pl
from jax.experimental.pallas import tpu as pltpu
from jax.experimental.pallas import tpu_sc as plsc

N_NODES = 10000
N_EDGES = 320000
D = 128
N_ASSETS = 512

NC = 2
NS = 16
CHUNK = 128
NCH = (-(-N_EDGES // (NC * NS * CHUNK)) + 3) // 4 * 4
EDGES_PER_WORKER = NCH * CHUNK
N_EDGES_PAD = NC * NS * EDGES_PER_WORKER
ACC_ROWS = 10240
ROWS_PER_SUB = ACC_ROWS // NS
OUT_BLK = 128

BLK = 1000


def _sc_scatter_body(y_hbm, eidx_hbm, out_hbm,
                     ibuf, rows, acc_sh, gsem, isem):
    c = lax.axis_index("c")
    s = lax.axis_index("s")

    pltpu.async_copy(eidx_hbm.at[c, s], ibuf, isem)

    def _zrow(r, carry):
        for k in range(D // 16):
            rows[r, pl.ds(k * 16, 16)] = jnp.zeros((16,), jnp.float32)
        return carry
    lax.fori_loop(0, OUT_BLK, _zrow, 0)
    for t in range(ROWS_PER_SUB // OUT_BLK):
        pltpu.sync_copy(rows,
                        acc_sh.at[pl.ds(s * ROWS_PER_SUB + t * OUT_BLK, OUT_BLK)])
    pltpu.make_async_copy(eidx_hbm.at[c, s], ibuf, isem).wait()
    plsc.subcore_barrier()

    for j in range(NCH):
        pltpu.sync_copy(y_hbm.at[ibuf.at[j, 0]], rows)
        pltpu.sync_copy(rows, acc_sh.at[ibuf.at[j, 1]], add=True)
    plsc.subcore_barrier()

    for t in range(ROWS_PER_SUB // OUT_BLK):
        r0 = s * ROWS_PER_SUB + t * OUT_BLK
        pltpu.sync_copy(acc_sh.at[pl.ds(r0, OUT_BLK)], rows)
        pltpu.sync_copy(rows, out_hbm.at[c, pl.ds(r0, OUT_BLK)])


@functools.cache
def _sc_scatter_kernel():
    return pl.kernel(
        _sc_scatter_body,
        out_type=jax.ShapeDtypeStruct((NC, ACC_ROWS, D), jnp.float32),
        mesh=plsc.VectorSubcoreMesh(core_axis_name="c", subcore_axis_name="s"),
        scratch_types=[
            pltpu.VMEM((NCH, 2, CHUNK), jnp.int32),
            pltpu.VMEM((CHUNK, D), jnp.float32),
            pltpu.VMEM_SHARED((ACC_ROWS, D), jnp.float32),
            *[pltpu.SemaphoreType.DMA for _ in range(2)],
        ],
    )


def _sc_scatter(y, eidx):
    return _sc_scatter_kernel()(y, eidx)


def _tc_pre_body(x_ref, wrel_ref, wroot_ref, y_ref, xr_ref):
    xb = x_ref[...]
    y_ref[...] = jnp.dot(xb, wrel_ref[...], preferred_element_type=jnp.float32)
    xr_ref[...] = jnp.dot(xb, wroot_ref[...], preferred_element_type=jnp.float32)


def _tc_pre(x, w_rel, w_root):
    return pl.pallas_call(
        _tc_pre_body,
        grid=(N_NODES // BLK,),
        in_specs=[
            pl.BlockSpec((BLK, D), lambda i: (i, 0)),
            pl.BlockSpec((D, D), lambda i: (0, 0)),
            pl.BlockSpec((D, D), lambda i: (0, 0)),
        ],
        out_specs=[
            pl.BlockSpec((BLK, D), lambda i: (i, 0)),
            pl.BlockSpec((BLK, D), lambda i: (i, 0)),
        ],
        out_shape=[
            jax.ShapeDtypeStruct((N_NODES, D), jnp.float32),
            jax.ShapeDtypeStruct((N_NODES, D), jnp.float32),
        ],
    )(x, w_rel, w_root)


def _tc_mid_body(agg_ref, xr_ref, b_ref, wrel_ref, wroot_ref, y_ref, hr_ref):
    h = jnp.maximum(agg_ref[0] + agg_ref[1] + xr_ref[...] + b_ref[...], 0.0)
    y_ref[...] = jnp.dot(h, wrel_ref[...], preferred_element_type=jnp.float32)
    hr_ref[...] = jnp.dot(h, wroot_ref[...], preferred_element_type=jnp.float32)


def _tc_mid(agg, xr, b, w_rel, w_root):
    return pl.pallas_call(
        _tc_mid_body,
        grid=(N_NODES // BLK,),
        in_specs=[
            pl.BlockSpec((NC, BLK, D), lambda i: (0, i, 0)),
            pl.BlockSpec((BLK, D), lambda i: (i, 0)),
            pl.BlockSpec((1, D), lambda i: (0, 0)),
            pl.BlockSpec((D, D), lambda i: (0, 0)),
            pl.BlockSpec((D, D), lambda i: (0, 0)),
        ],
        out_specs=[
            pl.BlockSpec((BLK, D), lambda i: (i, 0)),
            pl.BlockSpec((BLK, D), lambda i: (i, 0)),
        ],
        out_shape=[
            jax.ShapeDtypeStruct((N_NODES, D), jnp.float32),
            jax.ShapeDtypeStruct((N_NODES, D), jnp.float32),
        ],
    )(agg, xr, b, w_rel, w_root)


def _tc_colsum_body(agg_ref, hr_ref, b_ref, out_ref):
    i = pl.program_id(0)

    @pl.when(i == 0)
    def _():
        out_ref[...] = jnp.zeros_like(out_ref)

    h = jnp.maximum(agg_ref[0] + agg_ref[1] + hr_ref[...] + b_ref[...], 0.0)
    out_ref[...] += jnp.sum(h, axis=0, keepdims=True)


def _tc_colsum(agg, hr, b):
    return pl.pallas_call(
        _tc_colsum_body,
        grid=(N_NODES // BLK,),
        in_specs=[
            pl.BlockSpec((NC, BLK, D), lambda i: (0, i, 0)),
            pl.BlockSpec((BLK, D), lambda i: (i, 0)),
            pl.BlockSpec((1, D), lambda i: (0, 0)),
        ],
        out_specs=pl.BlockSpec((1, D), lambda i: (0, 0)),
        out_shape=jax.ShapeDtypeStruct((1, D), jnp.float32),
    )(agg, hr, b)


def _tc_head_body(cs_ref, w1_ref, b1_ref, w2_ref, b2_ref, out_ref):
    pooled = cs_ref[...] * (1.0 / N_NODES)
    o = jnp.maximum(jnp.dot(pooled, w1_ref[...], preferred_element_type=jnp.float32)
                    + b1_ref[...], 0.0)
    logits = jnp.dot(o, w2_ref[...], preferred_element_type=jnp.float32) + b2_ref[...]
    m = jnp.max(logits, axis=-1, keepdims=True)
    e = jnp.exp(logits - m)
    out_ref[...] = e / jnp.sum(e, axis=-1, keepdims=True)


def _tc_head(colsum, w1, b1, w2, b2):
    return pl.pallas_call(
        _tc_head_body,
        out_shape=jax.ShapeDtypeStruct((1, N_ASSETS), jnp.float32),
    )(colsum, w1, b1, w2, b2)


def kernel(x, edge_index, W1_rel, b1, W1_root, W2_rel, b2, W2_root,
           Wfc1, bfc1, Wfc2, bfc2):
    src = edge_index[0].astype(jnp.int32)
    dst = edge_index[1].astype(jnp.int32)
    npad = N_EDGES_PAD - N_EDGES
    srcp = jnp.concatenate([src, jnp.zeros((npad,), jnp.int32)]).reshape(
        NC, NS, NCH, CHUNK)
    dstp = jnp.concatenate([dst, jnp.full((npad,), N_NODES, jnp.int32)]).reshape(
        NC, NS, NCH, CHUNK)
    eidx = jnp.stack([srcp, dstp], axis=3)

    y1, xr1 = _tc_pre(x, W1_rel, W1_root)
    agg1 = _sc_scatter(y1, eidx)
    y2, hr2 = _tc_mid(agg1, xr1, b1.reshape(1, D), W2_rel, W2_root)
    agg2 = _sc_scatter(y2, eidx)
    colsum = _tc_colsum(agg2, hr2, b2.reshape(1, D))
    return _tc_head(colsum, Wfc1, bfc1.reshape(1, D), Wfc2, bfc2.reshape(1, N_ASSETS))


# speedup vs baseline: 1.1070x; 1.1070x over previous
import functools

import jax
import jax.numpy as jnp
from jax import lax
from jax.experimental import pallas as pl
from jax.experimental.pallas import tpu as pltpu
from jax.experimental.pallas import tpu_sc as plsc

N_NODES = 10000
N_EDGES = 320000
D = 128
N_ASSETS = 512

NC = 2
NS = 16
CHUNK = 128
NCH = (-(-N_EDGES // (NC * NS * CHUNK)) + 3) // 4 * 4
EDGES_PER_WORKER = NCH * CHUNK
N_EDGES_PAD = NC * NS * EDGES_PER_WORKER
ACC_ROWS = 10240
ROWS_PER_SUB = ACC_ROWS // NS
OUT_BLK = 128

BLK = 1000


def _sc_scatter_body(y_hbm, eidx_hbm, out_hbm,
                     ibuf, rows0, rows1, acc_sh,
                     gsem0, gsem1, is0, is1, is2, is3):
    rows = (rows0, rows1)
    gsems = (gsem0, gsem1)
    isems = (is0, is1, is2, is3)
    c = lax.axis_index("c")
    s = lax.axis_index("s")

    def _zrow(r, carry):
        for k in range(D // 16):
            rows0[r, pl.ds(k * 16, 16)] = jnp.zeros((16,), jnp.float32)
        return carry
    lax.fori_loop(0, OUT_BLK, _zrow, 0)
    for t in range(ROWS_PER_SUB // OUT_BLK):
        pltpu.sync_copy(rows0,
                        acc_sh.at[pl.ds(s * ROWS_PER_SUB + t * OUT_BLK, OUT_BLK)])
    plsc.subcore_barrier()

    def _ifetch(j, bk):
        pltpu.async_copy(eidx_hbm.at[c, s, j], ibuf.at[bk], isems[bk])

    def _iwait(j, bk):
        pltpu.make_async_copy(eidx_hbm.at[c, s, j], ibuf.at[bk], isems[bk]).wait()

    def _fire(bk, r):
        pltpu.async_copy(y_hbm.at[ibuf.at[bk, 0]], rows[r], gsems[r])

    def _drain(bk, r):
        pltpu.make_async_copy(y_hbm.at[ibuf.at[bk, 0]], rows[r], gsems[r]).wait()
        pltpu.sync_copy(rows[r], acc_sh.at[ibuf.at[bk, 1]], add=True)

    _ifetch(0, 0)
    _iwait(0, 0)
    _fire(0, 0)
    _ifetch(1, 1)
    _ifetch(2, 2)

    def _edge(g, carry):
        j = 4 * g
        for k in range(4):
            f = j + k + 1

            @pl.when(f < NCH)
            def _():
                _iwait(f, (k + 1) % 4)
                _fire((k + 1) % 4, (k + 1) % 2)

            _drain(k % 4, k % 2)

            p = j + k + 3

            @pl.when(p < NCH)
            def _():
                _ifetch(p, (k + 3) % 4)
        return carry
    lax.fori_loop(0, NCH // 4, _edge, 0)
    plsc.subcore_barrier()

    for t in range(ROWS_PER_SUB // OUT_BLK):
        r0 = s * ROWS_PER_SUB + t * OUT_BLK
        b = rows[t % 2]
        pltpu.sync_copy(acc_sh.at[pl.ds(r0, OUT_BLK)], b)
        pltpu.sync_copy(b, out_hbm.at[c, pl.ds(r0, OUT_BLK)])


@functools.cache
def _sc_scatter_kernel():
    return pl.kernel(
        _sc_scatter_body,
        out_type=jax.ShapeDtypeStruct((NC, ACC_ROWS, D), jnp.float32),
        mesh=plsc.VectorSubcoreMesh(core_axis_name="c", subcore_axis_name="s"),
        scratch_types=[
            pltpu.VMEM((4, 2, CHUNK), jnp.int32),
            pltpu.VMEM((CHUNK, D), jnp.float32),
            pltpu.VMEM((CHUNK, D), jnp.float32),
            pltpu.VMEM_SHARED((ACC_ROWS, D), jnp.float32),
            *[pltpu.SemaphoreType.DMA for _ in range(6)],
        ],
    )


def _sc_scatter(y, eidx):
    return _sc_scatter_kernel()(y, eidx)


def _tc_pre_body(x_ref, wrel_ref, wroot_ref, y_ref, xr_ref):
    xb = x_ref[...]
    y_ref[...] = jnp.dot(xb, wrel_ref[...], preferred_element_type=jnp.float32)
    xr_ref[...] = jnp.dot(xb, wroot_ref[...], preferred_element_type=jnp.float32)


def _tc_pre(x, w_rel, w_root):
    return pl.pallas_call(
        _tc_pre_body,
        grid=(N_NODES // BLK,),
        in_specs=[
            pl.BlockSpec((BLK, D), lambda i: (i, 0)),
            pl.BlockSpec((D, D), lambda i: (0, 0)),
            pl.BlockSpec((D, D), lambda i: (0, 0)),
        ],
        out_specs=[
            pl.BlockSpec((BLK, D), lambda i: (i, 0)),
            pl.BlockSpec((BLK, D), lambda i: (i, 0)),
        ],
        out_shape=[
            jax.ShapeDtypeStruct((N_NODES, D), jnp.float32),
            jax.ShapeDtypeStruct((N_NODES, D), jnp.float32),
        ],
    )(x, w_rel, w_root)


def _tc_mid_body(agg_ref, xr_ref, b_ref, wrel_ref, wroot_ref, y_ref, hr_ref):
    h = jnp.maximum(agg_ref[0] + agg_ref[1] + xr_ref[...] + b_ref[...], 0.0)
    y_ref[...] = jnp.dot(h, wrel_ref[...], preferred_element_type=jnp.float32)
    hr_ref[...] = jnp.dot(h, wroot_ref[...], preferred_element_type=jnp.float32)


def _tc_mid(agg, xr, b, w_rel, w_root):
    return pl.pallas_call(
        _tc_mid_body,
        grid=(N_NODES // BLK,),
        in_specs=[
            pl.BlockSpec((NC, BLK, D), lambda i: (0, i, 0)),
            pl.BlockSpec((BLK, D), lambda i: (i, 0)),
            pl.BlockSpec((1, D), lambda i: (0, 0)),
            pl.BlockSpec((D, D), lambda i: (0, 0)),
            pl.BlockSpec((D, D), lambda i: (0, 0)),
        ],
        out_specs=[
            pl.BlockSpec((BLK, D), lambda i: (i, 0)),
            pl.BlockSpec((BLK, D), lambda i: (i, 0)),
        ],
        out_shape=[
            jax.ShapeDtypeStruct((N_NODES, D), jnp.float32),
            jax.ShapeDtypeStruct((N_NODES, D), jnp.float32),
        ],
    )(agg, xr, b, w_rel, w_root)


def _tc_colsum_body(agg_ref, hr_ref, b_ref, out_ref):
    i = pl.program_id(0)

    @pl.when(i == 0)
    def _():
        out_ref[...] = jnp.zeros_like(out_ref)

    h = jnp.maximum(agg_ref[0] + agg_ref[1] + hr_ref[...] + b_ref[...], 0.0)
    out_ref[...] += jnp.sum(h, axis=0, keepdims=True)


def _tc_colsum(agg, hr, b):
    return pl.pallas_call(
        _tc_colsum_body,
        grid=(N_NODES // BLK,),
        in_specs=[
            pl.BlockSpec((NC, BLK, D), lambda i: (0, i, 0)),
            pl.BlockSpec((BLK, D), lambda i: (i, 0)),
            pl.BlockSpec((1, D), lambda i: (0, 0)),
        ],
        out_specs=pl.BlockSpec((1, D), lambda i: (0, 0)),
        out_shape=jax.ShapeDtypeStruct((1, D), jnp.float32),
    )(agg, hr, b)


def _tc_head_body(cs_ref, w1_ref, b1_ref, w2_ref, b2_ref, out_ref):
    pooled = cs_ref[...] * (1.0 / N_NODES)
    o = jnp.maximum(jnp.dot(pooled, w1_ref[...], preferred_element_type=jnp.float32)
                    + b1_ref[...], 0.0)
    logits = jnp.dot(o, w2_ref[...], preferred_element_type=jnp.float32) + b2_ref[...]
    m = jnp.max(logits, axis=-1, keepdims=True)
    e = jnp.exp(logits - m)
    out_ref[...] = e / jnp.sum(e, axis=-1, keepdims=True)


def _tc_head(colsum, w1, b1, w2, b2):
    return pl.pallas_call(
        _tc_head_body,
        out_shape=jax.ShapeDtypeStruct((1, N_ASSETS), jnp.float32),
    )(colsum, w1, b1, w2, b2)


def kernel(x, edge_index, W1_rel, b1, W1_root, W2_rel, b2, W2_root,
           Wfc1, bfc1, Wfc2, bfc2):
    src = edge_index[0].astype(jnp.int32)
    dst = edge_index[1].astype(jnp.int32)
    npad = N_EDGES_PAD - N_EDGES
    srcp = jnp.concatenate([src, jnp.zeros((npad,), jnp.int32)]).reshape(
        NC, NS, NCH, CHUNK)
    dstp = jnp.concatenate([dst, jnp.full((npad,), N_NODES, jnp.int32)]).reshape(
        NC, NS, NCH, CHUNK)
    eidx = jnp.stack([srcp, dstp], axis=3)

    y1, xr1 = _tc_pre(x, W1_rel, W1_root)
    agg1 = _sc_scatter(y1, eidx)
    y2, hr2 = _tc_mid(agg1, xr1, b1.reshape(1, D), W2_rel, W2_root)
    agg2 = _sc_scatter(y2, eidx)
    colsum = _tc_colsum(agg2, hr2, b2.reshape(1, D))
    return _tc_head(colsum, Wfc1, bfc1.reshape(1, D), Wfc2, bfc2.reshape(1, N_ASSETS))
